# baseline (speedup 1.0000x reference)
import functools

import jax
import jax.numpy as jnp
from jax import lax
from jax.experimental import pallas as pl
from jax.experimental.pallas import tpu as pltpu
from jax.experimental.pallas import tpu_sc as plsc

N = 10000
E = 160000
B = 64
K = 50
HDIM = 256
HALF = 128
EPS = 1e-5

NC = 2
NS = 16
CH = 128
NCHUNK = 80
NPHASE = 5
PCHUNK = NCHUNK // NPHASE
EPT = NCHUNK * CH
EPAD = NS * EPT
ACC_ROWS = 10240
STRIPE = ACC_ROWS // NS
OUT_STRIPE = N // NS


def _sc_agg_body(h2x, srcs, dsts, zrows, s_out,
                 src_v, dst_v, rows0, rows1, acc,
                 sem_g0, sem_g1, sem_s0, sem_s1):
  c = lax.axis_index("c")
  t = lax.axis_index("s")

  pltpu.sync_copy(zrows, acc.at[pl.ds(t * STRIPE, STRIPE)])
  plsc.subcore_barrier()

  def step2(i, carry):
    j0 = 2 * i
    j1 = j0 + 1
    pltpu.make_async_copy(h2x.at[src_v.at[0]], rows0, sem_g0).wait()
    pltpu.async_copy(rows0, acc.at[dst_v.at[j0]], sem_s0, add=True)
    pltpu.make_async_copy(h2x.at[src_v.at[0]], rows1, sem_g1).wait()
    pltpu.async_copy(rows1, acc.at[dst_v.at[j1]], sem_s1, add=True)

    @pl.when(j0 + 2 < PCHUNK)
    def _():
      pltpu.make_async_copy(rows0, acc.at[dst_v.at[0]], sem_s0).wait()
      pltpu.async_copy(h2x.at[src_v.at[j0 + 2]], rows0, sem_g0)

    @pl.when(j1 + 2 < PCHUNK)
    def _():
      pltpu.make_async_copy(rows1, acc.at[dst_v.at[0]], sem_s1).wait()
      pltpu.async_copy(h2x.at[src_v.at[j1 + 2]], rows1, sem_g1)
    return carry

  for p in range(NPHASE):
    pltpu.sync_copy(srcs.at[(c * NS + t) * NPHASE + p], src_v)
    pltpu.sync_copy(dsts.at[t * NPHASE + p], dst_v)
    pltpu.async_copy(h2x.at[src_v.at[0]], rows0, sem_g0)
    pltpu.async_copy(h2x.at[src_v.at[1]], rows1, sem_g1)
    lax.fori_loop(0, PCHUNK // 2, step2, 0)
    pltpu.make_async_copy(rows0, acc.at[dst_v.at[0]], sem_s0).wait()
    pltpu.make_async_copy(rows1, acc.at[dst_v.at[0]], sem_s1).wait()
  plsc.subcore_barrier()

  pltpu.sync_copy(acc.at[pl.ds(t * STRIPE, STRIPE)],
                  s_out.at[pl.ds(c * ACC_ROWS + t * STRIPE, STRIPE)])


def _make_sc_agg():
  mesh = plsc.VectorSubcoreMesh(core_axis_name="c", subcore_axis_name="s",
                                num_cores=NC, num_subcores=NS)
  scratch = [
      pltpu.VMEM((PCHUNK, CH), jnp.int32),
      pltpu.VMEM((PCHUNK, CH), jnp.int32),
      pltpu.VMEM((CH, HALF), jnp.float32),
      pltpu.VMEM((CH, HALF), jnp.float32),
      pltpu.VMEM_SHARED((ACC_ROWS, HALF), jnp.float32),
      pltpu.SemaphoreType.DMA,
      pltpu.SemaphoreType.DMA,
      pltpu.SemaphoreType.DMA,
      pltpu.SemaphoreType.DMA,
  ]
  return pl.kernel(
      _sc_agg_body,
      out_type=jax.ShapeDtypeStruct((NC * ACC_ROWS, HALF), jnp.float32),
      mesh=mesh,
      scratch_types=scratch,
  )


DNPHASE = 5
DPCHUNK = EPAD // (NC * NS * DNPHASE * CH)


def _sc_deg_body(dsts, zrows, ones_hbm, deg_out,
                 dst_v, ones_v, degacc, sem):
  c = lax.axis_index("c")
  t = lax.axis_index("s")
  pltpu.sync_copy(zrows, degacc.at[pl.ds(t * STRIPE, STRIPE)])
  pltpu.sync_copy(ones_hbm, ones_v)
  plsc.subcore_barrier()

  def fire(j, carry):
    pltpu.async_copy(ones_v, degacc.at[dst_v.at[j]], sem, add=True)
    return carry

  def drain(j, carry):
    pltpu.make_async_copy(ones_v, degacc.at[dst_v.at[0]], sem).wait()
    return carry

  for p in range(DNPHASE):
    pltpu.sync_copy(dsts.at[(c * NS + t) * DNPHASE + p], dst_v)
    lax.fori_loop(0, DPCHUNK, fire, 0)
    lax.fori_loop(0, DPCHUNK, drain, 0)
  plsc.subcore_barrier()

  pltpu.sync_copy(degacc.at[pl.ds(t * STRIPE, STRIPE)],
                  deg_out.at[pl.ds(c * ACC_ROWS + t * STRIPE, STRIPE)])


def _make_sc_deg():
  mesh = plsc.VectorSubcoreMesh(core_axis_name="c", subcore_axis_name="s",
                                num_cores=NC, num_subcores=NS)
  scratch = [
      pltpu.VMEM((DPCHUNK, CH), jnp.int32),
      pltpu.VMEM((CH, HALF), jnp.float32),
      pltpu.VMEM_SHARED((ACC_ROWS, HALF), jnp.float32),
      pltpu.SemaphoreType.DMA,
  ]
  return pl.kernel(
      _sc_deg_body,
      out_type=jax.ShapeDtypeStruct((NC * ACC_ROWS, HALF), jnp.float32),
      mesh=mesh,
      scratch_types=scratch,
  )


RB = 1000
NRB = N // RB


def _init_mm_body(x_ref, w_ref, b_ref, o_ref):
  h = jnp.dot(x_ref[...], w_ref[...], preferred_element_type=jnp.float32)
  o_ref[0] = h + b_ref[0]


def _init_matmul(x, w, b2):
  return pl.pallas_call(
      _init_mm_body,
      grid=(2, NRB),
      in_specs=[
          pl.BlockSpec((RB, 20), lambda c, i: (i, 0)),
          pl.BlockSpec((20, HALF), lambda c, i: (0, c)),
          pl.BlockSpec((1, 1, HALF), lambda c, i: (c, 0, 0)),
      ],
      out_specs=pl.BlockSpec((1, RB, HALF), lambda c, i: (c, i, 0)),
      out_shape=jax.ShapeDtypeStruct((2, N, HALF), jnp.float32),
  )(x, w, b2)


def _stats_body(h_ref, s_ref, deg_ref, u_ref, sum_ref, sq_ref):
  i = pl.program_id(0)
  deg = jnp.maximum(deg_ref[0, :, 0:1] + deg_ref[1, :, 0:1], 1.0)
  u = h_ref[...] + s_ref[...] / deg[None]
  u_ref[...] = u
  psum = jnp.sum(u, axis=1)
  psq = jnp.sum(u * u, axis=1)

  @pl.when(i == 0)
  def _():
    sum_ref[...] = psum
    sq_ref[...] = psq

  @pl.when(i > 0)
  def _():
    sum_ref[...] += psum
    sq_ref[...] += psq


def _stats(h, s, deg):
  return pl.pallas_call(
      _stats_body,
      grid=(NRB,),
      in_specs=[
          pl.BlockSpec((2, RB, HALF), lambda i: (0, i, 0)),
          pl.BlockSpec((2, RB, HALF), lambda i: (0, i, 0)),
          pl.BlockSpec((2, RB, HALF), lambda i: (0, i, 0)),
      ],
      out_specs=[
          pl.BlockSpec((2, RB, HALF), lambda i: (0, i, 0)),
          pl.BlockSpec((2, HALF), lambda i: (0, 0)),
          pl.BlockSpec((2, HALF), lambda i: (0, 0)),
      ],
      out_shape=[
          jax.ShapeDtypeStruct((2, N, HALF), jnp.float32),
          jax.ShapeDtypeStruct((2, HALF), jnp.float32),
          jax.ShapeDtypeStruct((2, HALF), jnp.float32),
      ],
  )(h, s, deg)


def _apply_body(u_ref, sum_ref, sq_ref, g_ref, b_ref, o_ref):
  m = sum_ref[...] / N
  v = sq_ref[...] / N - m * m
  inv = lax.rsqrt(v + EPS) * g_ref[...]
  o_ref[...] = jnp.maximum(
      (u_ref[...] - m[:, None, :]) * inv[:, None, :] + b_ref[...][:, None, :],
      0.0)


def _apply_bn(u, ssum, ssq, g, b):
  return pl.pallas_call(
      _apply_body,
      grid=(NRB,),
      in_specs=[
          pl.BlockSpec((2, RB, HALF), lambda i: (0, i, 0)),
          pl.BlockSpec((2, HALF), lambda i: (0, 0)),
          pl.BlockSpec((2, HALF), lambda i: (0, 0)),
          pl.BlockSpec((2, HALF), lambda i: (0, 0)),
          pl.BlockSpec((2, HALF), lambda i: (0, 0)),
      ],
      out_specs=pl.BlockSpec((2, RB, HALF), lambda i: (0, i, 0)),
      out_shape=jax.ShapeDtypeStruct((2, N, HALF), jnp.float32),
  )(u, ssum, ssq, g, b)


def _bn_rows(z, g, b):
  m = jnp.mean(z, axis=0, keepdims=True)
  v = jnp.mean(z * z, axis=0, keepdims=True) - m * m
  return (z - m) * lax.rsqrt(v + EPS) * g + b


def _head_body(u_ref, sum_ref, sq_ref, g2_ref, be2_ref, gid_ref,
               pg_ref, neigh_ref,
               wfc_ref, bfc_ref, wfc2_ref, bfc2_ref, wfc3_ref, bfc3_ref,
               wfc4_ref, bfc4_ref, gb_ref, bb_ref, gb2_ref, bb2_ref,
               gb3_ref, bb3_ref,
               preds_ref, h3_ref):
  m = sum_ref[...] / N
  v = sq_ref[...] / N - m * m
  inv = lax.rsqrt(v + EPS) * g2_ref[...]

  gid = gid_ref[...]
  onehot = (gid == lax.broadcasted_iota(jnp.int32, (N, B), 1)
            ).astype(jnp.float32)
  cnt = lax.dot_general(onehot, jnp.ones((N, 1), jnp.float32),
                        (((0,), (0,)), ((), ())),
                        preferred_element_type=jnp.float32)
  cnt = jnp.maximum(cnt, 1.0)

  qs = []
  for c in range(2):
    h2c = jnp.maximum(
        (u_ref[c] - m[c:c + 1]) * inv[c:c + 1] + be2_ref[...][c:c + 1], 0.0)
    qs.append(lax.dot_general(onehot, h2c, (((0,), (0,)), ((), ())),
                              preferred_element_type=jnp.float32))
  qemb = jnp.concatenate(qs, axis=1) / cnt

  wq = wfc_ref[0:HDIM]
  wp = wfc_ref[HDIM:2 * HDIM]
  wn = wfc_ref[2 * HDIM:3 * HDIM]
  z1 = (jnp.dot(qemb, wq, preferred_element_type=jnp.float32)
        + jnp.dot(pg_ref[...], wp, preferred_element_type=jnp.float32)
        + bfc_ref[...])
  zn = jnp.dot(neigh_ref[...], wn, preferred_element_type=jnp.float32)

  rk = lax.broadcasted_iota(jnp.int32, (B * K, B), 0) // K
  bk = lax.broadcasted_iota(jnp.int32, (B * K, B), 1)
  rep = (rk == bk).astype(jnp.float32)
  z = lax.dot_general(rep, z1, (((1,), (0,)), ((), ())),
                      preferred_element_type=jnp.float32) + zn

  h = jnp.maximum(_bn_rows(z, gb_ref[...], bb_ref[...]), 0.0)
  h = jnp.dot(h, wfc2_ref[...], preferred_element_type=jnp.float32) + bfc2_ref[...]
  h = jnp.maximum(_bn_rows(h, gb2_ref[...], bb2_ref[...]), 0.0)
  h = jnp.dot(h, wfc3_ref[...], preferred_element_type=jnp.float32) + bfc3_ref[...]
  h3 = jnp.maximum(_bn_rows(h, gb3_ref[...], bb3_ref[...]), 0.0)
  h3_ref[...] = h3
  logit = jnp.dot(h3, wfc4_ref[...], preferred_element_type=jnp.float32) + bfc4_ref[...]
  preds_ref[...] = jax.nn.sigmoid(logit)


def _head(u2, ssum2, ssq2, g2, be2, gid, pg, neigh,
          wfc, bfc, wfc2, bfc2, wfc3, bfc3, wfc4, bfc4,
          gb, bb, gb2, bb2, gb3, bb3):
  return pl.pallas_call(
      _head_body,
      out_shape=[
          jax.ShapeDtypeStruct((B * K, 1), jnp.float32),
          jax.ShapeDtypeStruct((B * K, HDIM), jnp.float32),
      ],
  )(u2, ssum2, ssq2, g2, be2, gid, pg, neigh,
    wfc, bfc, wfc2, bfc2, wfc3, bfc3, wfc4, bfc4,
    gb, bb, gb2, bb2, gb3, bb3)


def kernel(x, edge_index, graph_ids, pgNodeEmbList, neighEmbList,
           classWeightList, W_init, b_init, g1, be1, g2, be2,
           W_fc, b_fc, W_fc2, b_fc2, W_fc3, b_fc3, W_fc4, b_fc4,
           gb, bb, gb2, bb2, gb3, bb3):
  src = edge_index[0].astype(jnp.int32)
  dst = edge_index[1].astype(jnp.int32)
  pad = EPAD - E
  src = jnp.concatenate([src, jnp.zeros((pad,), jnp.int32)])
  dst = jnp.concatenate([dst, jnp.full((pad,), N, jnp.int32)])
  srcs = jnp.stack([src, src + N]).reshape(NC * NS * NPHASE, PCHUNK, CH)
  dsts = dst.reshape(NS * NPHASE, PCHUNK, CH)
  dsts_deg = dst.reshape(NC * NS * DNPHASE, DPCHUNK, CH)

  zrows = jnp.zeros((STRIPE, HALF), jnp.float32)
  ones_rows = jnp.ones((CH, HALF), jnp.float32)

  sc_agg = _make_sc_agg()
  sc_deg = _make_sc_deg()

  b2 = b_init.reshape(2, 1, HALF)
  h0 = _init_matmul(x, W_init, b2)

  deg = sc_deg(dsts_deg, zrows, ones_rows).reshape(NC, ACC_ROWS, HALF)
  s1 = sc_agg(h0.reshape(NC * N, HALF), srcs, dsts, zrows)
  u1, sum1, sq1 = _stats(h0, s1.reshape(NC, ACC_ROWS, HALF), deg)
  h1 = _apply_bn(u1, sum1, sq1, g1.reshape(2, HALF), be1.reshape(2, HALF))

  s2 = sc_agg(h1.reshape(NC * N, HALF), srcs, dsts, zrows)
  u2, sum2, sq2 = _stats(h1, s2.reshape(NC, ACC_ROWS, HALF), deg)

  preds, h3 = _head(
      u2, sum2, sq2, g2.reshape(2, HALF), be2.reshape(2, HALF),
      graph_ids.astype(jnp.int32).reshape(N, 1),
      pgNodeEmbList, neighEmbList,
      W_fc, b_fc.reshape(1, HDIM), W_fc2, b_fc2.reshape(1, HDIM),
      W_fc3, b_fc3.reshape(1, HDIM), W_fc4, b_fc4.reshape(1, 1),
      gb.reshape(1, HDIM), bb.reshape(1, HDIM),
      gb2.reshape(1, HDIM), bb2.reshape(1, HDIM),
      gb3.reshape(1, HDIM), bb3.reshape(1, HDIM))
  return preds.reshape(B, K), h3


# speedup vs baseline: 1.0302x; 1.0302x over previous
import functools

import jax
import jax.numpy as jnp
from jax import lax
from jax.experimental import pallas as pl
from jax.experimental.pallas import tpu as pltpu
from jax.experimental.pallas import tpu_sc as plsc

N = 10000
E = 160000
B = 64
K = 50
HDIM = 256
HALF = 128
EPS = 1e-5

NC = 2
NS = 16
CH = 128
NCHUNK = 80
NPHASE = 2
PCHUNK = NCHUNK // NPHASE
EPT = NCHUNK * CH
EPAD = NS * EPT
ACC_ROWS = 10240
STRIPE = ACC_ROWS // NS
OUT_STRIPE = N // NS


def _sc_agg_body(h2x, srcs, dsts, zrows, s_out,
                 src_v, dst_v, rows0, rows1, acc, sem_g0, sem_g1):
  c = lax.axis_index("c")
  t = lax.axis_index("s")

  pltpu.sync_copy(zrows, acc.at[pl.ds(t * STRIPE, STRIPE)])
  plsc.subcore_barrier()

  def step2(i, carry):
    j0 = 2 * i
    j1 = j0 + 1
    pltpu.make_async_copy(h2x.at[src_v.at[0]], rows0, sem_g0).wait()
    pltpu.async_copy(h2x.at[src_v.at[j1]], rows1, sem_g1)
    pltpu.sync_copy(rows0, acc.at[dst_v.at[j0]], add=True)
    pltpu.make_async_copy(h2x.at[src_v.at[0]], rows1, sem_g1).wait()

    @pl.when(j1 + 1 < PCHUNK)
    def _():
      pltpu.async_copy(h2x.at[src_v.at[j1 + 1]], rows0, sem_g0)
    pltpu.sync_copy(rows1, acc.at[dst_v.at[j1]], add=True)
    return carry

  for p in range(NPHASE):
    pltpu.sync_copy(srcs.at[(c * NS + t) * NPHASE + p], src_v)
    pltpu.sync_copy(dsts.at[t * NPHASE + p], dst_v)
    pltpu.async_copy(h2x.at[src_v.at[0]], rows0, sem_g0)
    lax.fori_loop(0, PCHUNK // 2, step2, 0)
  plsc.subcore_barrier()

  pltpu.sync_copy(acc.at[pl.ds(t * STRIPE, STRIPE)],
                  s_out.at[pl.ds(c * ACC_ROWS + t * STRIPE, STRIPE)])


def _make_sc_agg():
  mesh = plsc.VectorSubcoreMesh(core_axis_name="c", subcore_axis_name="s",
                                num_cores=NC, num_subcores=NS)
  scratch = [
      pltpu.VMEM((PCHUNK, CH), jnp.int32),
      pltpu.VMEM((PCHUNK, CH), jnp.int32),
      pltpu.VMEM((CH, HALF), jnp.float32),
      pltpu.VMEM((CH, HALF), jnp.float32),
      pltpu.VMEM_SHARED((ACC_ROWS, HALF), jnp.float32),
      pltpu.SemaphoreType.DMA,
      pltpu.SemaphoreType.DMA,
  ]
  return pl.kernel(
      _sc_agg_body,
      out_type=jax.ShapeDtypeStruct((NC * ACC_ROWS, HALF), jnp.float32),
      mesh=mesh,
      scratch_types=scratch,
  )


DNPHASE = 5
DPCHUNK = EPAD // (NC * NS * DNPHASE * CH)


def _sc_deg_body(dsts, zrows, ones_hbm, deg_out,
                 dst_v, ones_v, degacc, sem):
  c = lax.axis_index("c")
  t = lax.axis_index("s")
  pltpu.sync_copy(zrows, degacc.at[pl.ds(t * STRIPE, STRIPE)])
  pltpu.sync_copy(ones_hbm, ones_v)
  plsc.subcore_barrier()

  def fire(j, carry):
    pltpu.async_copy(ones_v, degacc.at[dst_v.at[j]], sem, add=True)
    return carry

  def drain(j, carry):
    pltpu.make_async_copy(ones_v, degacc.at[dst_v.at[0]], sem).wait()
    return carry

  for p in range(DNPHASE):
    pltpu.sync_copy(dsts.at[(c * NS + t) * DNPHASE + p], dst_v)
    lax.fori_loop(0, DPCHUNK, fire, 0)
    lax.fori_loop(0, DPCHUNK, drain, 0)
  plsc.subcore_barrier()

  pltpu.sync_copy(degacc.at[pl.ds(t * STRIPE, STRIPE)],
                  deg_out.at[pl.ds(c * ACC_ROWS + t * STRIPE, STRIPE)])


def _make_sc_deg():
  mesh = plsc.VectorSubcoreMesh(core_axis_name="c", subcore_axis_name="s",
                                num_cores=NC, num_subcores=NS)
  scratch = [
      pltpu.VMEM((DPCHUNK, CH), jnp.int32),
      pltpu.VMEM((CH, HALF), jnp.float32),
      pltpu.VMEM_SHARED((ACC_ROWS, HALF), jnp.float32),
      pltpu.SemaphoreType.DMA,
  ]
  return pl.kernel(
      _sc_deg_body,
      out_type=jax.ShapeDtypeStruct((NC * ACC_ROWS, HALF), jnp.float32),
      mesh=mesh,
      scratch_types=scratch,
  )


RB = 1000
NRB = N // RB


def _init_mm_body(x_ref, w_ref, b_ref, o_ref):
  h = jnp.dot(x_ref[...], w_ref[...], preferred_element_type=jnp.float32)
  o_ref[0] = h + b_ref[0]


def _init_matmul(x, w, b2):
  return pl.pallas_call(
      _init_mm_body,
      grid=(2, NRB),
      in_specs=[
          pl.BlockSpec((RB, 20), lambda c, i: (i, 0)),
          pl.BlockSpec((20, HALF), lambda c, i: (0, c)),
          pl.BlockSpec((1, 1, HALF), lambda c, i: (c, 0, 0)),
      ],
      out_specs=pl.BlockSpec((1, RB, HALF), lambda c, i: (c, i, 0)),
      out_shape=jax.ShapeDtypeStruct((2, N, HALF), jnp.float32),
  )(x, w, b2)


def _stats_body(h_ref, s_ref, deg_ref, u_ref, sum_ref, sq_ref):
  i = pl.program_id(0)
  deg = jnp.maximum(deg_ref[0, :, 0:1] + deg_ref[1, :, 0:1], 1.0)
  u = h_ref[...] + s_ref[...] / deg[None]
  u_ref[...] = u
  psum = jnp.sum(u, axis=1)
  psq = jnp.sum(u * u, axis=1)

  @pl.when(i == 0)
  def _():
    sum_ref[...] = psum
    sq_ref[...] = psq

  @pl.when(i > 0)
  def _():
    sum_ref[...] += psum
    sq_ref[...] += psq


def _stats(h, s, deg):
  return pl.pallas_call(
      _stats_body,
      grid=(NRB,),
      in_specs=[
          pl.BlockSpec((2, RB, HALF), lambda i: (0, i, 0)),
          pl.BlockSpec((2, RB, HALF), lambda i: (0, i, 0)),
          pl.BlockSpec((2, RB, HALF), lambda i: (0, i, 0)),
      ],
      out_specs=[
          pl.BlockSpec((2, RB, HALF), lambda i: (0, i, 0)),
          pl.BlockSpec((2, HALF), lambda i: (0, 0)),
          pl.BlockSpec((2, HALF), lambda i: (0, 0)),
      ],
      out_shape=[
          jax.ShapeDtypeStruct((2, N, HALF), jnp.float32),
          jax.ShapeDtypeStruct((2, HALF), jnp.float32),
          jax.ShapeDtypeStruct((2, HALF), jnp.float32),
      ],
  )(h, s, deg)


def _apply_body(u_ref, sum_ref, sq_ref, g_ref, b_ref, o_ref):
  m = sum_ref[...] / N
  v = sq_ref[...] / N - m * m
  inv = lax.rsqrt(v + EPS) * g_ref[...]
  o_ref[...] = jnp.maximum(
      (u_ref[...] - m[:, None, :]) * inv[:, None, :] + b_ref[...][:, None, :],
      0.0)


def _apply_bn(u, ssum, ssq, g, b):
  return pl.pallas_call(
      _apply_body,
      grid=(NRB,),
      in_specs=[
          pl.BlockSpec((2, RB, HALF), lambda i: (0, i, 0)),
          pl.BlockSpec((2, HALF), lambda i: (0, 0)),
          pl.BlockSpec((2, HALF), lambda i: (0, 0)),
          pl.BlockSpec((2, HALF), lambda i: (0, 0)),
          pl.BlockSpec((2, HALF), lambda i: (0, 0)),
      ],
      out_specs=pl.BlockSpec((2, RB, HALF), lambda i: (0, i, 0)),
      out_shape=jax.ShapeDtypeStruct((2, N, HALF), jnp.float32),
  )(u, ssum, ssq, g, b)


def _bn_rows(z, g, b):
  m = jnp.mean(z, axis=0, keepdims=True)
  v = jnp.mean(z * z, axis=0, keepdims=True) - m * m
  return (z - m) * lax.rsqrt(v + EPS) * g + b


def _head_body(u_ref, sum_ref, sq_ref, g2_ref, be2_ref, gid_ref,
               pg_ref, neigh_ref,
               wfc_ref, bfc_ref, wfc2_ref, bfc2_ref, wfc3_ref, bfc3_ref,
               wfc4_ref, bfc4_ref, gb_ref, bb_ref, gb2_ref, bb2_ref,
               gb3_ref, bb3_ref,
               preds_ref, h3_ref):
  m = sum_ref[...] / N
  v = sq_ref[...] / N - m * m
  inv = lax.rsqrt(v + EPS) * g2_ref[...]

  gid = gid_ref[...]
  onehot = (gid == lax.broadcasted_iota(jnp.int32, (N, B), 1)
            ).astype(jnp.float32)
  cnt = lax.dot_general(onehot, jnp.ones((N, 1), jnp.float32),
                        (((0,), (0,)), ((), ())),
                        preferred_element_type=jnp.float32)
  cnt = jnp.maximum(cnt, 1.0)

  qs = []
  for c in range(2):
    h2c = jnp.maximum(
        (u_ref[c] - m[c:c + 1]) * inv[c:c + 1] + be2_ref[...][c:c + 1], 0.0)
    qs.append(lax.dot_general(onehot, h2c, (((0,), (0,)), ((), ())),
                              preferred_element_type=jnp.float32))
  qemb = jnp.concatenate(qs, axis=1) / cnt

  wq = wfc_ref[0:HDIM]
  wp = wfc_ref[HDIM:2 * HDIM]
  wn = wfc_ref[2 * HDIM:3 * HDIM]
  z1 = (jnp.dot(qemb, wq, preferred_element_type=jnp.float32)
        + jnp.dot(pg_ref[...], wp, preferred_element_type=jnp.float32)
        + bfc_ref[...])
  zn = jnp.dot(neigh_ref[...], wn, preferred_element_type=jnp.float32)

  rk = lax.broadcasted_iota(jnp.int32, (B * K, B), 0) // K
  bk = lax.broadcasted_iota(jnp.int32, (B * K, B), 1)
  rep = (rk == bk).astype(jnp.float32)
  z = lax.dot_general(rep, z1, (((1,), (0,)), ((), ())),
                      preferred_element_type=jnp.float32) + zn

  h = jnp.maximum(_bn_rows(z, gb_ref[...], bb_ref[...]), 0.0)
  h = jnp.dot(h, wfc2_ref[...], preferred_element_type=jnp.float32) + bfc2_ref[...]
  h = jnp.maximum(_bn_rows(h, gb2_ref[...], bb2_ref[...]), 0.0)
  h = jnp.dot(h, wfc3_ref[...], preferred_element_type=jnp.float32) + bfc3_ref[...]
  h3 = jnp.maximum(_bn_rows(h, gb3_ref[...], bb3_ref[...]), 0.0)
  h3_ref[...] = h3
  logit = jnp.dot(h3, wfc4_ref[...], preferred_element_type=jnp.float32) + bfc4_ref[...]
  preds_ref[...] = jax.nn.sigmoid(logit)


def _head(u2, ssum2, ssq2, g2, be2, gid, pg, neigh,
          wfc, bfc, wfc2, bfc2, wfc3, bfc3, wfc4, bfc4,
          gb, bb, gb2, bb2, gb3, bb3):
  return pl.pallas_call(
      _head_body,
      out_shape=[
          jax.ShapeDtypeStruct((B * K, 1), jnp.float32),
          jax.ShapeDtypeStruct((B * K, HDIM), jnp.float32),
      ],
  )(u2, ssum2, ssq2, g2, be2, gid, pg, neigh,
    wfc, bfc, wfc2, bfc2, wfc3, bfc3, wfc4, bfc4,
    gb, bb, gb2, bb2, gb3, bb3)


def kernel(x, edge_index, graph_ids, pgNodeEmbList, neighEmbList,
           classWeightList, W_init, b_init, g1, be1, g2, be2,
           W_fc, b_fc, W_fc2, b_fc2, W_fc3, b_fc3, W_fc4, b_fc4,
           gb, bb, gb2, bb2, gb3, bb3):
  src = edge_index[0].astype(jnp.int32)
  dst = edge_index[1].astype(jnp.int32)
  pad = EPAD - E
  src = jnp.concatenate([src, jnp.zeros((pad,), jnp.int32)])
  dst = jnp.concatenate([dst, jnp.full((pad,), N, jnp.int32)])
  srcs = jnp.stack([src, src + N]).reshape(NC * NS * NPHASE, PCHUNK, CH)
  dsts = dst.reshape(NS * NPHASE, PCHUNK, CH)
  dsts_deg = dst.reshape(NC * NS * DNPHASE, DPCHUNK, CH)

  zrows = jnp.zeros((STRIPE, HALF), jnp.float32)
  ones_rows = jnp.ones((CH, HALF), jnp.float32)

  sc_agg = _make_sc_agg()
  sc_deg = _make_sc_deg()

  b2 = b_init.reshape(2, 1, HALF)
  h0 = _init_matmul(x, W_init, b2)

  deg = sc_deg(dsts_deg, zrows, ones_rows).reshape(NC, ACC_ROWS, HALF)
  s1 = sc_agg(h0.reshape(NC * N, HALF), srcs, dsts, zrows)
  u1, sum1, sq1 = _stats(h0, s1.reshape(NC, ACC_ROWS, HALF), deg)
  h1 = _apply_bn(u1, sum1, sq1, g1.reshape(2, HALF), be1.reshape(2, HALF))

  s2 = sc_agg(h1.reshape(NC * N, HALF), srcs, dsts, zrows)
  u2, sum2, sq2 = _stats(h1, s2.reshape(NC, ACC_ROWS, HALF), deg)

  preds, h3 = _head(
      u2, sum2, sq2, g2.reshape(2, HALF), be2.reshape(2, HALF),
      graph_ids.astype(jnp.int32).reshape(N, 1),
      pgNodeEmbList, neighEmbList,
      W_fc, b_fc.reshape(1, HDIM), W_fc2, b_fc2.reshape(1, HDIM),
      W_fc3, b_fc3.reshape(1, HDIM), W_fc4, b_fc4.reshape(1, 1),
      gb.reshape(1, HDIM), bb.reshape(1, HDIM),
      gb2.reshape(1, HDIM), bb2.reshape(1, HDIM),
      gb3.reshape(1, HDIM), bb3.reshape(1, HDIM))
  return preds.reshape(B, K), h3


# speedup vs baseline: 1.0318x; 1.0015x over previous
import jax
import jax.numpy as jnp
from jax import lax
from jax.experimental import pallas as pl
from jax.experimental.pallas import tpu as pltpu
from jax.experimental.pallas import tpu_sc as plsc

N = 10000
E = 160000
B = 64
K = 50
HDIM = 256
HALF = 128
EPS = 1e-5

NC = 2
NS = 16
CH = 128
NCHUNK = 80
NPHASE = 2
PCHUNK = NCHUNK // NPHASE
EPT = NCHUNK * CH
EPAD = NS * EPT
ACC_ROWS = 10240
STRIPE = ACC_ROWS // NS


def _sc_agg_body(h2x, srcs, dsts, zrows, s_out,
                 src_v, dst_v, rows0, rows1, acc, sem_g0, sem_g1):
  c = lax.axis_index("c")
  t = lax.axis_index("s")

  pltpu.sync_copy(zrows, acc.at[pl.ds(t * STRIPE, STRIPE)])
  plsc.subcore_barrier()

  def step2(i, carry):
    j0 = 2 * i
    j1 = j0 + 1
    pltpu.make_async_copy(h2x.at[src_v.at[0]], rows0, sem_g0).wait()
    pltpu.async_copy(h2x.at[src_v.at[j1]], rows1, sem_g1)
    pltpu.sync_copy(rows0, acc.at[dst_v.at[j0]], add=True)
    pltpu.make_async_copy(h2x.at[src_v.at[0]], rows1, sem_g1).wait()

    @pl.when(j1 + 1 < PCHUNK)
    def _():
      pltpu.async_copy(h2x.at[src_v.at[j1 + 1]], rows0, sem_g0)
    pltpu.sync_copy(rows1, acc.at[dst_v.at[j1]], add=True)
    return carry

  for p in range(NPHASE):
    pltpu.sync_copy(srcs.at[(c * NS + t) * NPHASE + p], src_v)
    pltpu.sync_copy(dsts.at[t * NPHASE + p], dst_v)
    pltpu.async_copy(h2x.at[src_v.at[0]], rows0, sem_g0)
    lax.fori_loop(0, PCHUNK // 2, step2, 0)
  plsc.subcore_barrier()

  pltpu.sync_copy(acc.at[pl.ds(t * STRIPE, STRIPE)],
                  s_out.at[pl.ds(c * ACC_ROWS + t * STRIPE, STRIPE)])


def _make_sc_agg():
  mesh = plsc.VectorSubcoreMesh(core_axis_name="c", subcore_axis_name="s",
                                num_cores=NC, num_subcores=NS)
  scratch = [
      pltpu.VMEM((PCHUNK, CH), jnp.int32),
      pltpu.VMEM((PCHUNK, CH), jnp.int32),
      pltpu.VMEM((CH, HALF), jnp.float32),
      pltpu.VMEM((CH, HALF), jnp.float32),
      pltpu.VMEM_SHARED((ACC_ROWS, HALF), jnp.float32),
      pltpu.SemaphoreType.DMA,
      pltpu.SemaphoreType.DMA,
  ]
  return pl.kernel(
      _sc_agg_body,
      out_type=jax.ShapeDtypeStruct((NC * ACC_ROWS, HALF), jnp.float32),
      mesh=mesh,
      scratch_types=scratch,
  )


DNPHASE = 5
DPCHUNK = EPAD // (NC * NS * DNPHASE * CH)


def _sc_deg_body(dsts, zrows, ones_hbm, deg_out,
                 dst_v, ones_v, degacc, sem):
  c = lax.axis_index("c")
  t = lax.axis_index("s")
  pltpu.sync_copy(zrows, degacc.at[pl.ds(t * STRIPE, STRIPE)])
  pltpu.sync_copy(ones_hbm, ones_v)
  plsc.subcore_barrier()

  def fire(j, carry):
    pltpu.async_copy(ones_v, degacc.at[dst_v.at[j]], sem, add=True)
    return carry

  def drain(j, carry):
    pltpu.make_async_copy(ones_v, degacc.at[dst_v.at[0]], sem).wait()
    return carry

  for p in range(DNPHASE):
    pltpu.sync_copy(dsts.at[(c * NS + t) * DNPHASE + p], dst_v)
    lax.fori_loop(0, DPCHUNK, fire, 0)
    lax.fori_loop(0, DPCHUNK, drain, 0)
  plsc.subcore_barrier()

  pltpu.sync_copy(degacc.at[pl.ds(t * STRIPE, STRIPE)],
                  deg_out.at[pl.ds(c * ACC_ROWS + t * STRIPE, STRIPE)])


def _make_sc_deg():
  mesh = plsc.VectorSubcoreMesh(core_axis_name="c", subcore_axis_name="s",
                                num_cores=NC, num_subcores=NS)
  scratch = [
      pltpu.VMEM((DPCHUNK, CH), jnp.int32),
      pltpu.VMEM((CH, HALF), jnp.float32),
      pltpu.VMEM_SHARED((ACC_ROWS, HALF), jnp.float32),
      pltpu.SemaphoreType.DMA,
  ]
  return pl.kernel(
      _sc_deg_body,
      out_type=jax.ShapeDtypeStruct((NC * ACC_ROWS, HALF), jnp.float32),
      mesh=mesh,
      scratch_types=scratch,
  )


RB = 1000
NRB = N // RB


def _init_mm_body(x_ref, w_ref, b_ref, o_ref):
  h = jnp.dot(x_ref[...], w_ref[...], preferred_element_type=jnp.float32)
  o_ref[0] = h + b_ref[0]


def _init_matmul(x, w, b2):
  return pl.pallas_call(
      _init_mm_body,
      grid=(2, NRB),
      in_specs=[
          pl.BlockSpec((RB, 20), lambda c, i: (i, 0)),
          pl.BlockSpec((20, HALF), lambda c, i: (0, c)),
          pl.BlockSpec((1, 1, HALF), lambda c, i: (c, 0, 0)),
      ],
      out_specs=pl.BlockSpec((1, RB, HALF), lambda c, i: (c, i, 0)),
      out_shape=jax.ShapeDtypeStruct((2, N, HALF), jnp.float32),
  )(x, w, b2)


def _stats_body(h_ref, s_ref, deg_ref, u_ref, sum_ref, sq_ref):
  i = pl.program_id(0)
  deg = jnp.maximum(deg_ref[0, :, 0:1] + deg_ref[1, :, 0:1], 1.0)
  u = h_ref[...] + s_ref[...] / deg[None]
  u_ref[...] = u
  psum = jnp.sum(u, axis=1)
  psq = jnp.sum(u * u, axis=1)

  @pl.when(i == 0)
  def _():
    sum_ref[...] = psum
    sq_ref[...] = psq

  @pl.when(i > 0)
  def _():
    sum_ref[...] += psum
    sq_ref[...] += psq


def _stats(h, s, deg):
  return pl.pallas_call(
      _stats_body,
      grid=(NRB,),
      in_specs=[
          pl.BlockSpec((2, RB, HALF), lambda i: (0, i, 0)),
          pl.BlockSpec((2, RB, HALF), lambda i: (0, i, 0)),
          pl.BlockSpec((2, RB, HALF), lambda i: (0, i, 0)),
      ],
      out_specs=[
          pl.BlockSpec((2, RB, HALF), lambda i: (0, i, 0)),
          pl.BlockSpec((2, HALF), lambda i: (0, 0)),
          pl.BlockSpec((2, HALF), lambda i: (0, 0)),
      ],
      out_shape=[
          jax.ShapeDtypeStruct((2, N, HALF), jnp.float32),
          jax.ShapeDtypeStruct((2, HALF), jnp.float32),
          jax.ShapeDtypeStruct((2, HALF), jnp.float32),
      ],
  )(h, s, deg)


def _apply_body(u_ref, sum_ref, sq_ref, g_ref, b_ref, o_ref):
  m = sum_ref[...] / N
  v = sq_ref[...] / N - m * m
  inv = lax.rsqrt(v + EPS) * g_ref[...]
  o_ref[...] = jnp.maximum(
      (u_ref[...] - m[:, None, :]) * inv[:, None, :] + b_ref[...][:, None, :],
      0.0)


def _apply_bn(u, ssum, ssq, g, b):
  return pl.pallas_call(
      _apply_body,
      grid=(NRB,),
      in_specs=[
          pl.BlockSpec((2, RB, HALF), lambda i: (0, i, 0)),
          pl.BlockSpec((2, HALF), lambda i: (0, 0)),
          pl.BlockSpec((2, HALF), lambda i: (0, 0)),
          pl.BlockSpec((2, HALF), lambda i: (0, 0)),
          pl.BlockSpec((2, HALF), lambda i: (0, 0)),
      ],
      out_specs=pl.BlockSpec((2, RB, HALF), lambda i: (0, i, 0)),
      out_shape=jax.ShapeDtypeStruct((2, N, HALF), jnp.float32),
  )(u, ssum, ssq, g, b)


def _bn_rows(z, g, b):
  m = jnp.mean(z, axis=0, keepdims=True)
  v = jnp.mean(z * z, axis=0, keepdims=True) - m * m
  return (z - m) * lax.rsqrt(v + EPS) * g + b


def _head_body(u_ref, sum_ref, sq_ref, g2_ref, be2_ref, gid_ref,
               pg_ref, neigh_ref,
               wfc_ref, bfc_ref, wfc2_ref, bfc2_ref, wfc3_ref, bfc3_ref,
               wfc4_ref, bfc4_ref, gb_ref, bb_ref, gb2_ref, bb2_ref,
               gb3_ref, bb3_ref,
               preds_ref, h3_ref):
  m = sum_ref[...] / N
  v = sq_ref[...] / N - m * m
  inv = lax.rsqrt(v + EPS) * g2_ref[...]

  gid = gid_ref[...]
  onehot = (gid == lax.broadcasted_iota(jnp.int32, (N, B), 1)
            ).astype(jnp.float32)
  cnt = lax.dot_general(onehot, jnp.ones((N, 1), jnp.float32),
                        (((0,), (0,)), ((), ())),
                        preferred_element_type=jnp.float32)
  cnt = jnp.maximum(cnt, 1.0)

  qs = []
  for c in range(2):
    h2c = jnp.maximum(
        (u_ref[c] - m[c:c + 1]) * inv[c:c + 1] + be2_ref[...][c:c + 1], 0.0)
    qs.append(lax.dot_general(onehot, h2c, (((0,), (0,)), ((), ())),
                              preferred_element_type=jnp.float32))
  qemb = jnp.concatenate(qs, axis=1) / cnt

  wq = wfc_ref[0:HDIM]
  wp = wfc_ref[HDIM:2 * HDIM]
  wn = wfc_ref[2 * HDIM:3 * HDIM]
  z1 = (jnp.dot(qemb, wq, preferred_element_type=jnp.float32)
        + jnp.dot(pg_ref[...], wp, preferred_element_type=jnp.float32)
        + bfc_ref[...])
  zn = jnp.dot(neigh_ref[...], wn, preferred_element_type=jnp.float32)

  rk = lax.broadcasted_iota(jnp.int32, (B * K, B), 0) // K
  bk = lax.broadcasted_iota(jnp.int32, (B * K, B), 1)
  rep = (rk == bk).astype(jnp.float32)
  z = lax.dot_general(rep, z1, (((1,), (0,)), ((), ())),
                      preferred_element_type=jnp.float32) + zn

  h = jnp.maximum(_bn_rows(z, gb_ref[...], bb_ref[...]), 0.0)
  h = jnp.dot(h, wfc2_ref[...], preferred_element_type=jnp.float32) + bfc2_ref[...]
  h = jnp.maximum(_bn_rows(h, gb2_ref[...], bb2_ref[...]), 0.0)
  h = jnp.dot(h, wfc3_ref[...], preferred_element_type=jnp.float32) + bfc3_ref[...]
  h3 = jnp.maximum(_bn_rows(h, gb3_ref[...], bb3_ref[...]), 0.0)
  h3_ref[...] = h3
  logit = jnp.dot(h3, wfc4_ref[...], preferred_element_type=jnp.float32) + bfc4_ref[...]
  preds_ref[...] = jax.nn.sigmoid(logit)


def _head(u2, ssum2, ssq2, g2, be2, gid, pg, neigh,
          wfc, bfc, wfc2, bfc2, wfc3, bfc3, wfc4, bfc4,
          gb, bb, gb2, bb2, gb3, bb3):
  return pl.pallas_call(
      _head_body,
      out_shape=[
          jax.ShapeDtypeStruct((B * K, 1), jnp.float32),
          jax.ShapeDtypeStruct((B * K, HDIM), jnp.float32),
      ],
  )(u2, ssum2, ssq2, g2, be2, gid, pg, neigh,
    wfc, bfc, wfc2, bfc2, wfc3, bfc3, wfc4, bfc4,
    gb, bb, gb2, bb2, gb3, bb3)


def kernel(x, edge_index, graph_ids, pgNodeEmbList, neighEmbList,
           classWeightList, W_init, b_init, g1, be1, g2, be2,
           W_fc, b_fc, W_fc2, b_fc2, W_fc3, b_fc3, W_fc4, b_fc4,
           gb, bb, gb2, bb2, gb3, bb3):
  src = edge_index[0].astype(jnp.int32)
  dst = edge_index[1].astype(jnp.int32)
  pad = EPAD - E
  src = jnp.concatenate([src, jnp.zeros((pad,), jnp.int32)])
  dst = jnp.concatenate([dst, jnp.full((pad,), N, jnp.int32)])
  srcs = jnp.stack([src, src + N]).reshape(NC * NS * NPHASE, PCHUNK, CH)
  dsts = dst.reshape(NS * NPHASE, PCHUNK, CH)
  dsts_deg = dst.reshape(NC * NS * DNPHASE, DPCHUNK, CH)

  zrows = jnp.zeros((STRIPE, HALF), jnp.float32)
  ones_rows = jnp.ones((CH, HALF), jnp.float32)

  sc_agg = _make_sc_agg()
  sc_deg = _make_sc_deg()

  b2 = b_init.reshape(2, 1, HALF)
  h0 = _init_matmul(x, W_init, b2)

  deg = sc_deg(dsts_deg, zrows, ones_rows).reshape(NC, ACC_ROWS, HALF)
  s1 = sc_agg(h0.reshape(NC * N, HALF), srcs, dsts, zrows)
  u1, sum1, sq1 = _stats(h0, s1.reshape(NC, ACC_ROWS, HALF), deg)
  h1 = _apply_bn(u1, sum1, sq1, g1.reshape(2, HALF), be1.reshape(2, HALF))

  s2 = sc_agg(h1.reshape(NC * N, HALF), srcs, dsts, zrows)
  u2, sum2, sq2 = _stats(h1, s2.reshape(NC, ACC_ROWS, HALF), deg)

  preds, h3 = _head(
      u2, sum2, sq2, g2.reshape(2, HALF), be2.reshape(2, HALF),
      graph_ids.astype(jnp.int32).reshape(N, 1),
      pgNodeEmbList, neighEmbList,
      W_fc, b_fc.reshape(1, HDIM), W_fc2, b_fc2.reshape(1, HDIM),
      W_fc3, b_fc3.reshape(1, HDIM), W_fc4, b_fc4.reshape(1, 1),
      gb.reshape(1, HDIM), bb.reshape(1, HDIM),
      gb2.reshape(1, HDIM), bb2.reshape(1, HDIM),
      gb3.reshape(1, HDIM), bb3.reshape(1, HDIM))
  return preds.reshape(B, K), h3
